# baseline (speedup 1.0000x reference)
import jax
import jax.numpy as jnp
from jax.experimental import pallas as pl


def _gs3d(vol, grid):
    Bv, C, Dd, Hh, Ww = vol.shape
    x, y, z = grid[..., 0], grid[..., 1], grid[..., 2]
    ix = (x + 1.0) * 0.5 * (Ww - 1)
    iy = (y + 1.0) * 0.5 * (Hh - 1)
    iz = (z + 1.0) * 0.5 * (Dd - 1)
    ix0f, iy0f, iz0f = jnp.floor(ix), jnp.floor(iy), jnp.floor(iz)
    wx1, wy1, wz1 = ix - ix0f, iy - iy0f, iz - iz0f
    wx0, wy0, wz0 = 1.0 - wx1, 1.0 - wy1, 1.0 - wz1
    x0 = ix0f.astype(jnp.int32); x1 = x0 + 1
    y0 = iy0f.astype(jnp.int32); y1 = y0 + 1
    z0 = iz0f.astype(jnp.int32); z1 = z0 + 1
    vol_flat = vol.reshape(Bv, C, Dd * Hh * Ww)

    def corner(xc, yc, zc, w):
        mask = ((xc >= 0) & (xc < Ww) & (yc >= 0) & (yc < Hh) & (zc >= 0) & (zc < Dd)).astype(vol.dtype)
        xcc = jnp.clip(xc, 0, Ww - 1)
        ycc = jnp.clip(yc, 0, Hh - 1)
        zcc = jnp.clip(zc, 0, Dd - 1)
        idx = (zcc * Hh + ycc) * Ww + xcc
        vals = jnp.take_along_axis(vol_flat, idx[:, None, :], axis=2)
        return vals * (mask * w)[:, None, :]

    return (corner(x0, y0, z0, wx0 * wy0 * wz0)
            + corner(x1, y0, z0, wx1 * wy0 * wz0)
            + corner(x0, y1, z0, wx0 * wy1 * wz0)
            + corner(x1, y1, z0, wx1 * wy1 * wz0)
            + corner(x0, y0, z1, wx0 * wy0 * wz1)
            + corner(x1, y0, z1, wx1 * wy0 * wz1)
            + corner(x0, y1, z1, wx0 * wy1 * wz1)
            + corner(x1, y1, z1, wx1 * wy1 * wz1))


def _identity_kernel(x_ref, o_ref):
    o_ref[...] = x_ref[...]


def kernel(origins, directions, lengths, densities, features, world2local):
    Bb, nr = origins.shape[0], origins.shape[1]
    pp = lengths.shape[-1]
    ones = jnp.ones(origins.shape[:-1] + (1,), dtype=origins.dtype)
    o_h = jnp.concatenate([origins, ones], axis=-1)
    o_loc = jnp.einsum('bnk,bkj->bnj', o_h, world2local)
    o_loc = o_loc[..., :3] / o_loc[..., 3:4]
    d_loc = jnp.einsum('bnk,bkj->bnj', directions, world2local[:, :3, :3])
    pts = o_loc[:, :, None, :] + d_loc[:, :, None, :] * lengths[..., None]
    pts_flat = pts.reshape(Bb, -1, 3)
    dens = _gs3d(densities, pts_flat)
    feat = _gs3d(features, pts_flat)
    rd = jnp.transpose(dens, (0, 2, 1)).reshape(Bb, nr, pp, densities.shape[1])
    rf = jnp.transpose(feat, (0, 2, 1)).reshape(Bb, nr, pp, features.shape[1])
    rd2 = rd.reshape(4096, 128)
    rd2 = pl.pallas_call(
        _identity_kernel,
        out_shape=jax.ShapeDtypeStruct(rd2.shape, rd2.dtype),
    )(rd2)
    rd = rd2.reshape(rd.shape)
    return (rd, rf)


# speedup vs baseline: 1.3297x; 1.3297x over previous
import functools

import jax
import jax.numpy as jnp
from jax import lax
from jax.experimental import pallas as pl
from jax.experimental.pallas import tpu as pltpu
from jax.experimental.pallas import tpu_sc as plsc

NC, NS, L = 2, 16, 16
NW = NC * NS


def _make_sc_sampler(B, NR, P, D, H, W, CF):
    N = B * NR * P
    NRAYS = B * NR
    RPT = NRAYS // NW
    PPT = RPT * P
    CP = 128
    GROUPS = CP // L
    RAYS_PER_CHUNK = CP // P
    GROUPS_PER_RAY = P // L
    CHUNKS = PPT // CP
    CC = 1 + CF
    DHW = D * H * W
    assert NRAYS % NW == 0 and P % L == 0 and CP % P == 0 and PPT % CP == 0

    mesh = plsc.VectorSubcoreMesh(core_axis_name="c", subcore_axis_name="s")

    @functools.partial(
        pl.kernel,
        mesh=mesh,
        compiler_params=pltpu.CompilerParams(
            needs_layout_passes=False, use_tc_tiling_on_sc=False),
        out_type=(
            jax.ShapeDtypeStruct((N,), jnp.float32),
            jax.ShapeDtypeStruct((N, CF), jnp.float32),
        ),
        scratch_types=[
            pltpu.VMEM((RPT,), jnp.float32),
            pltpu.VMEM((RPT,), jnp.float32),
            pltpu.VMEM((RPT,), jnp.float32),
            pltpu.VMEM((RPT,), jnp.float32),
            pltpu.VMEM((RPT,), jnp.float32),
            pltpu.VMEM((RPT,), jnp.float32),
            pltpu.VMEM((CP,), jnp.float32),
            pltpu.VMEM((8, CP), jnp.int32),
            pltpu.VMEM((8, GROUPS, L), jnp.float32),
            pltpu.VMEM((8, CP, 16), jnp.float32),
            pltpu.VMEM((CP,), jnp.float32),
            pltpu.VMEM((CP, CF), jnp.float32),
            pltpu.SemaphoreType.DMA,
        ],
    )
    def sampler(table_h, ox_h, oy_h, oz_h, dx_h, dy_h, dz_h, t_h,
                dens_h, feat_h,
                oxv, oyv, ozv, dxv, dyv, dzv, tv, idxv, wv, rowsv,
                densv, featv, gsem):
        cid = lax.axis_index("c")
        sid = lax.axis_index("s")
        wid = sid * NC + cid
        ray_base = wid * RPT
        pt_base = wid * PPT

        pltpu.sync_copy(ox_h.at[pl.ds(ray_base, RPT)], oxv)
        pltpu.sync_copy(oy_h.at[pl.ds(ray_base, RPT)], oyv)
        pltpu.sync_copy(oz_h.at[pl.ds(ray_base, RPT)], ozv)
        pltpu.sync_copy(dx_h.at[pl.ds(ray_base, RPT)], dxv)
        pltpu.sync_copy(dy_h.at[pl.ds(ray_base, RPT)], dyv)
        pltpu.sync_copy(dz_h.at[pl.ds(ray_base, RPT)], dzv)

        iota = lax.iota(jnp.int32, L)
        fone = jnp.full((L,), 1.0, jnp.float32)
        fzero = jnp.full((L,), 0.0, jnp.float32)

        def axis_setup(pval, extent):
            gc = (pval + 1.0) * (0.5 * (extent - 1))
            ti = gc.astype(jnp.int32)
            tf = ti.astype(jnp.float32)
            neg = (gc < tf)
            lo = ti - neg.astype(jnp.int32)
            lof = tf - neg.astype(jnp.float32)
            fr = gc - lof
            w_lo = fone - fr
            w_hi = fr
            v_lo = (lo >= 0) & (lo <= extent - 1)
            v_hi = (lo >= -1) & (lo <= extent - 2)
            w_lo = jnp.where(v_lo, w_lo, fzero)
            w_hi = jnp.where(v_hi, w_hi, fzero)
            lo_c = jnp.clip(lo, 0, extent - 1)
            hi_c = jnp.clip(lo + 1, 0, extent - 1)
            return w_lo, w_hi, lo_c, hi_c

        def chunk_body(ci, carry):
            pltpu.sync_copy(t_h.at[pl.ds(pt_base + ci * CP, CP)], tv)
            for g in range(GROUPS):
                ray_l = ci * RAYS_PER_CHUNK + (g // GROUPS_PER_RAY)
                ridx = jnp.full((L,), ray_l, jnp.int32)
                oxs = plsc.load_gather(oxv, [ridx])
                oys = plsc.load_gather(oyv, [ridx])
                ozs = plsc.load_gather(ozv, [ridx])
                dxs = plsc.load_gather(dxv, [ridx])
                dys = plsc.load_gather(dyv, [ridx])
                dzs = plsc.load_gather(dzv, [ridx])
                t16 = tv[pl.ds(g * L, L)]
                px = oxs + dxs * t16
                py = oys + dys * t16
                pz = ozs + dzs * t16
                wx0, wx1, x0, x1 = axis_setup(px, W)
                wy0, wy1, y0, y1 = axis_setup(py, H)
                wz0, wz1, z0, z1 = axis_setup(pz, D)
                b_off = jnp.full((L,), 0, jnp.int32) + (ray_base // NR) * DHW
                z0t = z0 * (H * W) + b_off
                z1t = z1 * (H * W) + b_off
                y0t = y0 * W
                y1t = y1 * W
                wxy00 = wx0 * wy0
                wxy10 = wx1 * wy0
                wxy01 = wx0 * wy1
                wxy11 = wx1 * wy1
                corners = (
                    (z0t + y0t + x0, wxy00 * wz0),
                    (z0t + y0t + x1, wxy10 * wz0),
                    (z0t + y1t + x0, wxy01 * wz0),
                    (z0t + y1t + x1, wxy11 * wz0),
                    (z1t + y0t + x0, wxy00 * wz1),
                    (z1t + y0t + x1, wxy10 * wz1),
                    (z1t + y1t + x0, wxy01 * wz1),
                    (z1t + y1t + x1, wxy11 * wz1),
                )
                for k, (vk, wk) in enumerate(corners):
                    idxv[k, pl.ds(g * L, L)] = vk
                    wv[k, g, :] = wk
            handles = [
                pltpu.async_copy(table_h.at[idxv.at[k]], rowsv.at[k], gsem)
                for k in range(8)
            ]
            for h in handles:
                h.wait()
            for g in range(GROUPS):
                p_idx = iota + (g * L)
                accs = [None] * CC
                for k in range(8):
                    wk = wv[k, g, :]
                    kvec = jnp.full((L,), k, jnp.int32)
                    for c in range(CC):
                        cvec = jnp.full((L,), c, jnp.int32)
                        val = plsc.load_gather(rowsv, [kvec, p_idx, cvec])
                        contrib = wk * val
                        accs[c] = contrib if accs[c] is None else accs[c] + contrib
                densv[pl.ds(g * L, L)] = accs[0]
                for c in range(1, CC):
                    plsc.store_scatter(featv, [p_idx, jnp.full((L,), c - 1, jnp.int32)],
                                       accs[c])
            pltpu.sync_copy(densv, dens_h.at[pl.ds(pt_base + ci * CP, CP)])
            pltpu.sync_copy(featv, feat_h.at[pl.ds(pt_base + ci * CP, CP)])
            return carry

        lax.fori_loop(0, CHUNKS, chunk_body, 0)

    return sampler


def kernel(origins, directions, lengths, densities, features, world2local):
    B, NR, _ = origins.shape
    P = lengths.shape[-1]
    _, CD, D, H, W = densities.shape
    CF = features.shape[1]

    ones = jnp.ones(origins.shape[:-1] + (1,), dtype=origins.dtype)
    o_h = jnp.concatenate([origins, ones], axis=-1)
    o_loc = jnp.einsum('bnk,bkj->bnj', o_h, world2local)
    o_loc = o_loc[..., :3] / o_loc[..., 3:4]
    d_loc = jnp.einsum('bnk,bkj->bnj', directions, world2local[:, :3, :3])

    ox = o_loc[..., 0].reshape(-1)
    oy = o_loc[..., 1].reshape(-1)
    oz = o_loc[..., 2].reshape(-1)
    dx = d_loc[..., 0].reshape(-1)
    dy = d_loc[..., 1].reshape(-1)
    dz = d_loc[..., 2].reshape(-1)
    tflat = lengths.reshape(-1)

    volc = jnp.concatenate([densities, features], axis=1)
    table = jnp.transpose(volc.reshape(B, 1 + CF, D * H * W), (0, 2, 1))
    table = jnp.pad(table, ((0, 0), (0, 0), (0, 16 - (1 + CF))))
    table = table.reshape(B * D * H * W, 16)

    sampler = _make_sc_sampler(B, NR, P, D, H, W, CF)
    dens_flat, feat_flat = sampler(table, ox, oy, oz, dx, dy, dz, tflat)
    rd = dens_flat.reshape(B, NR, P, 1)
    rf = feat_flat.reshape(B, NR, P, CF)
    return (rd, rf)


# speedup vs baseline: 1.7165x; 1.2908x over previous
import functools

import jax
import jax.numpy as jnp
from jax import lax
from jax.experimental import pallas as pl
from jax.experimental.pallas import tpu as pltpu
from jax.experimental.pallas import tpu_sc as plsc

NC, NS, L = 2, 16, 16
NW = NC * NS


def _make_sc_sampler(B, NR, P, D, H, W, CF):
    N = B * NR * P
    NRAYS = B * NR
    RPT = NRAYS // NW
    PPT = RPT * P
    CP = 128
    GROUPS = CP // L
    RAYS_PER_CHUNK = CP // P
    GROUPS_PER_RAY = P // L
    CHUNKS = PPT // CP
    CC = 1 + CF
    DHW = D * H * W
    assert NRAYS % NW == 0 and P % L == 0 and CP % P == 0 and PPT % CP == 0

    mesh = plsc.VectorSubcoreMesh(core_axis_name="c", subcore_axis_name="s")

    @functools.partial(
        pl.kernel,
        mesh=mesh,
        compiler_params=pltpu.CompilerParams(
            needs_layout_passes=False, use_tc_tiling_on_sc=False),
        out_type=(
            jax.ShapeDtypeStruct((N,), jnp.float32),
            jax.ShapeDtypeStruct((N, CF), jnp.float32),
        ),
        scratch_types=[
            pltpu.VMEM((RPT,), jnp.float32),
            pltpu.VMEM((RPT,), jnp.float32),
            pltpu.VMEM((RPT,), jnp.float32),
            pltpu.VMEM((RPT,), jnp.float32),
            pltpu.VMEM((RPT,), jnp.float32),
            pltpu.VMEM((RPT,), jnp.float32),
            pltpu.VMEM((CP,), jnp.float32),
            pltpu.VMEM((8, CP), jnp.int32),
            pltpu.VMEM((8, GROUPS, L), jnp.float32),
            pltpu.VMEM((8, CP, 16), jnp.float32),
            pltpu.VMEM((CP,), jnp.float32),
            pltpu.VMEM((CP, CF), jnp.float32),
            pltpu.SemaphoreType.DMA,
        ],
    )
    def sampler(table_h, ox_h, oy_h, oz_h, dx_h, dy_h, dz_h, t_h,
                dens_h, feat_h,
                oxv, oyv, ozv, dxv, dyv, dzv, tv, idxv, wv, rowsv,
                densv, featv, gsem):
        cid = lax.axis_index("c")
        sid = lax.axis_index("s")
        wid = sid * NC + cid
        ray_base = wid * RPT
        pt_base = wid * PPT

        pltpu.sync_copy(ox_h.at[pl.ds(ray_base, RPT)], oxv)
        pltpu.sync_copy(oy_h.at[pl.ds(ray_base, RPT)], oyv)
        pltpu.sync_copy(oz_h.at[pl.ds(ray_base, RPT)], ozv)
        pltpu.sync_copy(dx_h.at[pl.ds(ray_base, RPT)], dxv)
        pltpu.sync_copy(dy_h.at[pl.ds(ray_base, RPT)], dyv)
        pltpu.sync_copy(dz_h.at[pl.ds(ray_base, RPT)], dzv)

        iota = lax.iota(jnp.int32, L)
        fone = jnp.full((L,), 1.0, jnp.float32)
        fzero = jnp.full((L,), 0.0, jnp.float32)

        def axis_setup(pval, extent):
            gc = (pval + 1.0) * (0.5 * (extent - 1))
            ti = gc.astype(jnp.int32)
            tf = ti.astype(jnp.float32)
            neg = (gc < tf)
            lo = ti - neg.astype(jnp.int32)
            lof = tf - neg.astype(jnp.float32)
            fr = gc - lof
            w_lo = fone - fr
            w_hi = fr
            v_lo = (lo >= 0) & (lo <= extent - 1)
            v_hi = (lo >= -1) & (lo <= extent - 2)
            w_lo = jnp.where(v_lo, w_lo, fzero)
            w_hi = jnp.where(v_hi, w_hi, fzero)
            lo_c = jnp.clip(lo, 0, extent - 1)
            hi_c = jnp.clip(lo + 1, 0, extent - 1)
            return w_lo, w_hi, lo_c, hi_c

        def chunk_body(ci, carry):
            pltpu.sync_copy(t_h.at[pl.ds(pt_base + ci * CP, CP)], tv)
            for g in range(GROUPS):
                ray_l = ci * RAYS_PER_CHUNK + (g // GROUPS_PER_RAY)
                ridx = jnp.full((L,), ray_l, jnp.int32)
                oxs = plsc.load_gather(oxv, [ridx])
                oys = plsc.load_gather(oyv, [ridx])
                ozs = plsc.load_gather(ozv, [ridx])
                dxs = plsc.load_gather(dxv, [ridx])
                dys = plsc.load_gather(dyv, [ridx])
                dzs = plsc.load_gather(dzv, [ridx])
                t16 = tv[pl.ds(g * L, L)]
                px = oxs + dxs * t16
                py = oys + dys * t16
                pz = ozs + dzs * t16
                wx0, wx1, x0, x1 = axis_setup(px, W)
                wy0, wy1, y0, y1 = axis_setup(py, H)
                wz0, wz1, z0, z1 = axis_setup(pz, D)
                b_off = jnp.full((L,), 0, jnp.int32) + (ray_base // NR) * DHW
                z0t = z0 * (H * W) + b_off
                z1t = z1 * (H * W) + b_off
                y0t = y0 * W
                y1t = y1 * W
                wxy00 = wx0 * wy0
                wxy10 = wx1 * wy0
                wxy01 = wx0 * wy1
                wxy11 = wx1 * wy1
                corners = (
                    (z0t + y0t + x0, wxy00 * wz0),
                    (z0t + y0t + x1, wxy10 * wz0),
                    (z0t + y1t + x0, wxy01 * wz0),
                    (z0t + y1t + x1, wxy11 * wz0),
                    (z1t + y0t + x0, wxy00 * wz1),
                    (z1t + y0t + x1, wxy10 * wz1),
                    (z1t + y1t + x0, wxy01 * wz1),
                    (z1t + y1t + x1, wxy11 * wz1),
                )
                for k, (vk, wk) in enumerate(corners):
                    idxv[k, pl.ds(g * L, L)] = vk
                    wv[k, g, :] = wk
            handles = [
                pltpu.async_copy(table_h.at[idxv.at[k]], rowsv.at[k], gsem)
                for k in range(8)
            ]
            for h in handles:
                h.wait()
            for g in range(GROUPS):
                p_idx = iota + (g * L)
                accs = [None] * CC
                for k in range(8):
                    wk = wv[k, g, :]
                    kvec = jnp.full((L,), k, jnp.int32)
                    for c in range(CC):
                        cvec = jnp.full((L,), c, jnp.int32)
                        val = plsc.load_gather(rowsv, [kvec, p_idx, cvec])
                        contrib = wk * val
                        accs[c] = contrib if accs[c] is None else accs[c] + contrib
                densv[pl.ds(g * L, L)] = accs[0]
                for c in range(1, CC):
                    plsc.store_scatter(featv, [p_idx, jnp.full((L,), c - 1, jnp.int32)],
                                       accs[c])
            pltpu.sync_copy(densv, dens_h.at[pl.ds(pt_base + ci * CP, CP)])
            pltpu.sync_copy(featv, feat_h.at[pl.ds(pt_base + ci * CP, CP)])
            return carry

        lax.fori_loop(0, CHUNKS, chunk_body, 0)

    return sampler


def kernel(origins, directions, lengths, densities, features, world2local):
    B, NR, _ = origins.shape
    P = lengths.shape[-1]
    _, CD, D, H, W = densities.shape
    CF = features.shape[1]

    ones = jnp.ones(origins.shape[:-1] + (1,), dtype=origins.dtype)
    o_h = jnp.concatenate([origins, ones], axis=-1)
    o_loc = jnp.einsum('bnk,bkj->bnj', o_h, world2local)
    o_loc = o_loc[..., :3] / o_loc[..., 3:4]
    d_loc = jnp.einsum('bnk,bkj->bnj', directions, world2local[:, :3, :3])

    ox = o_loc[..., 0].reshape(-1)
    oy = o_loc[..., 1].reshape(-1)
    oz = o_loc[..., 2].reshape(-1)
    dx = d_loc[..., 0].reshape(-1)
    dy = d_loc[..., 1].reshape(-1)
    dz = d_loc[..., 2].reshape(-1)
    tflat = lengths.reshape(-1)

    volc = jnp.concatenate([densities, features], axis=1)
    eye = jnp.eye(1 + CF, 16, dtype=jnp.float32)
    table = jnp.einsum('bkv,kc->bvc', volc.reshape(B, 1 + CF, D * H * W), eye,
                       preferred_element_type=jnp.float32)
    table = table.reshape(B * D * H * W, 16)

    sampler = _make_sc_sampler(B, NR, P, D, H, W, CF)
    dens_flat, feat_flat = sampler(table, ox, oy, oz, dx, dy, dz, tflat)
    rd = dens_flat.reshape(B, NR, P, 1)
    rf = feat_flat.reshape(B, NR, P, CF)
    return (rd, rf)
